# baseline (speedup 1.0000x reference)
import functools

import jax
import jax.numpy as jnp
from jax import lax
from jax.experimental import pallas as pl
from jax.experimental.pallas import tpu as pltpu
from jax.experimental.pallas import tpu_sc as plsc

_DIM = 64
_TAB = 4096
_B = 16384

_CHUNK = 128


def _prep_body(xr_ref, table_ref, idx_ref):
    xv = xr_ref[...]
    ml = jnp.floor(jnp.max(xv)) + jnp.float32(1.0)
    idx_ref[...] = ((xv / ml) * (ml - jnp.float32(1.0))).astype(jnp.int32)
    c = lax.broadcasted_iota(jnp.int32, (_TAB // 2, 2 * _DIM), 1)
    m = lax.broadcasted_iota(jnp.int32, (_TAB // 2, 2 * _DIM), 0)
    pos = (2 * m + (c // _DIM)).astype(jnp.float32)
    col = c % _DIM
    dt = jnp.exp((col & ~1).astype(jnp.float32) * (-jnp.log(ml) / _DIM))
    ang = pos * dt
    table_ref[...] = jnp.where((col & 1) == 0, jnp.sin(ang), jnp.cos(ang))


_prep = pl.pallas_call(
    _prep_body,
    out_shape=(
        jax.ShapeDtypeStruct((_TAB // 2, 2 * _DIM), jnp.float32),
        jax.ShapeDtypeStruct((_B // 128, 256), jnp.int32),
    ),
)


@functools.cache
def _make_gather_add():
    info = plsc.get_sparse_core_info()
    nc, ns, nl = info.num_cores, info.num_subcores, info.num_lanes
    nw = nc * ns
    rpw = _B // nw
    gpw = 2 * rpw
    nchunk = gpw // _CHUNK

    @functools.partial(
        pl.kernel,
        mesh=plsc.VectorSubcoreMesh(core_axis_name="c", subcore_axis_name="s"),
        out_type=jax.ShapeDtypeStruct((_B, _DIM), jnp.float32),
        scratch_types=[
            pltpu.VMEM((nchunk, _CHUNK), jnp.int32),
            pltpu.VMEM((gpw, _DIM), jnp.float32),
            pltpu.VMEM((rpw, _DIM), jnp.float32),
            pltpu.SemaphoreType.DMA,
        ],
        compiler_params=pltpu.CompilerParams(use_tc_tiling_on_sc=False),
    )
    def _gather_add(table_hbm, idx_hbm, out_hbm, idx_v, g_v, o_v, sem):
        wid = lax.axis_index("s") * nc + lax.axis_index("c")
        pltpu.sync_copy(idx_hbm.at[pl.ds(wid * nchunk, nchunk)], idx_v)
        copies = []
        for j in range(nchunk):
            copies.append(
                pltpu.async_copy(
                    table_hbm.at[idx_v.at[j]],
                    g_v.at[pl.ds(j * _CHUNK, _CHUNK)],
                    sem,
                )
            )
        for cp in copies:
            cp.wait()

        def _row(k, carry):
            for cc in range(_DIM // nl):
                sl = pl.ds(cc * nl, nl)
                o_v[k, sl] = g_v[2 * k, sl] + g_v[2 * k + 1, sl]
            return carry

        lax.fori_loop(0, rpw, _row, 0)
        pltpu.sync_copy(o_v, out_hbm.at[pl.ds(wid * rpw, rpw)])

    return _gather_add


def kernel(x):
    table2, idx2 = _prep(x.reshape(_B // 128, 256))
    table = table2.reshape(_TAB, _DIM)
    idx = idx2.reshape(_B * 2 // _CHUNK, _CHUNK)
    return _make_gather_add()(table, idx)


# speedup vs baseline: 1.0533x; 1.0533x over previous
import functools

import jax
import jax.numpy as jnp
from jax import lax
from jax.experimental import pallas as pl
from jax.experimental.pallas import tpu as pltpu
from jax.experimental.pallas import tpu_sc as plsc

_DIM = 64
_TAB = 4096
_B = 16384

_CHUNK = 128


def _prep_body(xr_ref, table_ref, idx_ref):
    xv = xr_ref[...]
    ml = jnp.floor(jnp.max(xv)) + jnp.float32(1.0)
    idx_ref[...] = ((xv / ml) * (ml - jnp.float32(1.0))).astype(jnp.int32)
    c = lax.broadcasted_iota(jnp.int32, (_TAB // 2, 2 * _DIM), 1)
    m = lax.broadcasted_iota(jnp.int32, (_TAB // 2, 2 * _DIM), 0)
    pos = (2 * m + (c // _DIM)).astype(jnp.float32)
    col = c % _DIM
    dt = jnp.exp((col & ~1).astype(jnp.float32) * (-jnp.log(ml) / _DIM))
    ang = pos * dt
    table_ref[...] = jnp.where((col & 1) == 0, jnp.sin(ang), jnp.cos(ang))


_prep = pl.pallas_call(
    _prep_body,
    out_shape=(
        jax.ShapeDtypeStruct((_TAB // 2, 2 * _DIM), jnp.float32),
        jax.ShapeDtypeStruct((_B * 2 // 128, 128), jnp.int32),
    ),
)


@functools.cache
def _make_gather_add():
    info = plsc.get_sparse_core_info()
    nc, ns, nl = info.num_cores, info.num_subcores, info.num_lanes
    nw = nc * ns
    rpw = _B // nw
    gpw = 2 * rpw
    nchunk = gpw // _CHUNK

    opw = _CHUNK // 2

    @functools.partial(
        pl.kernel,
        mesh=plsc.VectorSubcoreMesh(core_axis_name="c", subcore_axis_name="s"),
        out_type=jax.ShapeDtypeStruct((_B, _DIM), jnp.float32),
        scratch_types=[
            pltpu.VMEM((nchunk, _CHUNK), jnp.int32),
            pltpu.VMEM((gpw, _DIM), jnp.float32),
            pltpu.VMEM((rpw, _DIM), jnp.float32),
            [pltpu.SemaphoreType.DMA] * 8,
            pltpu.SemaphoreType.DMA,
        ],
        compiler_params=pltpu.CompilerParams(use_tc_tiling_on_sc=False),
    )
    def _gather_add(table_hbm, idx_hbm, out_hbm, idx_v, g_v, o_v, gsems, osem):
        wid = lax.axis_index("s") * nc + lax.axis_index("c")
        pltpu.sync_copy(idx_hbm.at[pl.ds(wid * nchunk, nchunk)], idx_v)
        gathers = [
            pltpu.async_copy(
                table_hbm.at[idx_v.at[j]],
                g_v.at[pl.ds(j * _CHUNK, _CHUNK)],
                gsems[j],
            )
            for j in range(nchunk)
        ]
        base = wid * rpw
        writes = []
        for j in range(nchunk):
            gathers[j].wait()

            def _row(k, carry):
                for cc in range(_DIM // nl):
                    sl = pl.ds(cc * nl, nl)
                    o_v[k, sl] = g_v[2 * k, sl] + g_v[2 * k + 1, sl]
                return carry

            lax.fori_loop(j * opw, (j + 1) * opw, _row, 0)
            writes.append(
                pltpu.async_copy(
                    o_v.at[pl.ds(j * opw, opw)],
                    out_hbm.at[pl.ds(base + j * opw, opw)],
                    osem,
                )
            )
        for wr in writes:
            wr.wait()

    return _gather_add


def kernel(x):
    table2, idx = _prep(x.reshape(_B * 2 // 128, 128))
    table = table2.reshape(_TAB, _DIM)
    return _make_gather_add()(table, idx)
